# baseline (speedup 1.0000x reference)
import functools

import jax
import jax.numpy as jnp
from jax import lax
from jax.experimental import pallas as pl
from jax.experimental.pallas import tpu as pltpu
from jax.experimental.pallas import tpu_sc as plsc


def _make_sc_transpose(vocab, hid):
  info = plsc.get_sparse_core_info()
  nc, ns = info.num_cores, info.num_subcores
  nw = nc * ns
  lanes = 2 * hid
  nb_full = vocab // lanes
  tail = vocab - nb_full * lanes
  max_iters = (nb_full + nw - 1) // nw

  mesh = plsc.VectorSubcoreMesh(core_axis_name="c", subcore_axis_name="s")

  @functools.partial(
      pl.kernel,
      mesh=mesh,
      compiler_params=pltpu.CompilerParams(needs_layout_passes=False),
      out_type=jax.ShapeDtypeStruct((vocab * hid,), jnp.float32),
      scratch_types=[
          pltpu.VMEM((hid, lanes), jnp.float32),
          pltpu.VMEM((hid, lanes), jnp.float32),
          pltpu.VMEM((hid * lanes,), jnp.float32),
          pltpu.VMEM((hid * lanes,), jnp.float32),
          pltpu.VMEM((hid, hid), jnp.float32),
          pltpu.SemaphoreType.DMA,
          pltpu.SemaphoreType.DMA,
          pltpu.SemaphoreType.DMA,
          pltpu.SemaphoreType.DMA,
          pltpu.SemaphoreType.DMA,
      ],
  )
  def sc_t(tin, out, vbuf0, vbuf1, tbuf0, tbuf1, vtail,
           in0, in1, out0, out1, tsem):
    wid = lax.axis_index("s") * nc + lax.axis_index("c")
    n_my = (nb_full - wid + nw - 1) // nw
    vbufs = (vbuf0, vbuf1)
    tbufs = (tbuf0, tbuf1)
    in_sems = (in0, in1)
    out_sems = (out0, out1)
    f16 = lax.iota(jnp.int32, 16)

    def start_in(i):
      g = wid + i * nw
      b = lax.rem(i, 2)
      for bb in range(2):
        @pl.when(b == bb)
        def _():
          pltpu.async_copy(tin.at[:, pl.ds(g * lanes, lanes)],
                           vbufs[bb], in_sems[bb])

    def transpose_block(bb, n_cols, src, dst):
      def col(v, carry):
        vv = jnp.zeros((16,), jnp.int32) + v
        for kk in range(hid // 16):
          vec = plsc.load_gather(src, [f16 + kk * 16, vv])
          dst[pl.ds(v * hid + kk * 16, 16)] = vec
        return carry
      lax.fori_loop(0, n_cols, col, 0)

    @pl.when(n_my > 0)
    def _():
      start_in(0)

      def body(i, carry):
        g = wid + i * nw
        b = lax.rem(i, 2)
        for bb in range(2):
          @pl.when(b == bb)
          def _():
            pltpu.make_async_copy(tin.at[:, pl.ds(g * lanes, lanes)],
                                  vbufs[bb], in_sems[bb]).wait()

        @pl.when(i + 1 < n_my)
        def _():
          start_in(i + 1)

        for bb in range(2):
          @pl.when(b == bb)
          def _():
            @pl.when(i >= 2)
            def _():
              pltpu.make_async_copy(
                  tbufs[bb], out.at[pl.ds(g * hid * lanes, hid * lanes)],
                  out_sems[bb]).wait()
            transpose_block(bb, lanes, vbufs[bb], tbufs[bb])
            pltpu.async_copy(
                tbufs[bb], out.at[pl.ds(g * hid * lanes, hid * lanes)],
                out_sems[bb])
        return carry

      lax.fori_loop(0, n_my, body, 0)

      def drain(i, carry):
        b = lax.rem(i, 2)
        g = wid + i * nw
        for bb in range(2):
          @pl.when((b == bb) & (i + 2 >= n_my))
          def _():
            pltpu.make_async_copy(
                tbufs[bb], out.at[pl.ds(g * hid * lanes, hid * lanes)],
                out_sems[bb]).wait()
        return carry
      lax.fori_loop(jnp.maximum(n_my - 2, 0), n_my, drain, 0)

    if tail:
      @pl.when(wid == nw - 1)
      def _():
        pltpu.async_copy(tin.at[:, pl.ds(nb_full * lanes, tail)], vtail, tsem)
        pltpu.make_async_copy(tin.at[:, pl.ds(nb_full * lanes, tail)], vtail,
                              tsem).wait()
        def col(v, carry):
          vv = jnp.zeros((16,), jnp.int32) + v
          for kk in range(hid // 16):
            vec = plsc.load_gather(vtail, [f16 + kk * 16, vv])
            tbuf0[pl.ds(v * hid + kk * 16, 16)] = vec
          return carry
        lax.fori_loop(0, tail, col, 0)
        pltpu.async_copy(
            tbuf0.at[pl.ds(0, tail * hid)],
            out.at[pl.ds(nb_full * lanes * hid, tail * hid)], tsem)
        pltpu.make_async_copy(
            tbuf0.at[pl.ds(0, tail * hid)],
            out.at[pl.ds(nb_full * lanes * hid, tail * hid)], tsem).wait()

  return sc_t


def _make_sc_gather(vocab, hid, n_rows, chunk):
  info = plsc.get_sparse_core_info()
  nc, ns = info.num_cores, info.num_subcores
  nw = nc * ns
  per_w = n_rows // nw
  assert n_rows % nw == 0 and per_w % chunk == 0
  n_chunks = per_w // chunk

  mesh = plsc.VectorSubcoreMesh(core_axis_name="c", subcore_axis_name="s")

  @functools.partial(
      pl.kernel,
      mesh=mesh,
      compiler_params=pltpu.CompilerParams(use_tc_tiling_on_sc=False),
      out_type=jax.ShapeDtypeStruct((n_rows, 2 * hid), jnp.float32),
      scratch_types=[
          pltpu.VMEM((chunk,), jnp.int32),
          pltpu.VMEM((chunk, hid), jnp.float32),
          pltpu.SemaphoreType.DMA,
      ],
  )
  def sc_gather(table_hbm, idx_hbm, out_hbm, idx_v, rows_v, sem):
    wid = lax.axis_index("s") * nc + lax.axis_index("c")
    w_base = wid * per_w

    def body(i, carry):
      base = w_base + i * chunk
      pltpu.sync_copy(idx_hbm.at[pl.ds(base, chunk)], idx_v)
      pltpu.async_copy(table_hbm.at[idx_v], rows_v, sem).wait()
      pltpu.sync_copy(rows_v, out_hbm.at[pl.ds(base, chunk), pl.ds(0, hid)])
      return carry

    lax.fori_loop(0, n_chunks, body, 0)

  return sc_gather


def _conv_body(x_ref, w_ref, b_ref, out_ref):
  bs, s, hid = x_ref.shape
  x = x_ref[...]
  zero = jnp.zeros((bs, 1, hid), jnp.float32)
  x_prev = jnp.concatenate([zero, x[:, :-1, :]], axis=1)
  x_next = jnp.concatenate([x[:, 1:, :], zero], axis=1)
  xcat = jnp.concatenate([x_prev, x, x_next], axis=2)
  y = jnp.dot(
      xcat.reshape(bs * s, 3 * hid), w_ref[...],
      preferred_element_type=jnp.float32)
  y = y.reshape(bs, s, hid)
  m = jnp.max(y, axis=1)
  out_ref[...] = jnp.maximum(m + b_ref[...], 0.0)


def _make_tc_conv(b, s, hid, bs_blk):
  assert b % bs_blk == 0
  grid = (b // bs_blk,)
  return pl.pallas_call(
      _conv_body,
      grid=grid,
      in_specs=[
          pl.BlockSpec((bs_blk, s, hid), lambda i: (i, 0, 0)),
          pl.BlockSpec((3 * hid, hid), lambda i: (0, 0)),
          pl.BlockSpec((1, hid), lambda i: (0, 0)),
      ],
      out_specs=pl.BlockSpec((bs_blk, hid), lambda i: (i, 0)),
      out_shape=jax.ShapeDtypeStruct((b, hid), jnp.float32),
  )


def kernel(input, table, conv_w, conv_b):
  b, s = input.shape
  vocab, hid = table.shape
  k = conv_w.shape[2]
  n_rows = b * s

  idx = input.reshape(n_rows)

  sc_t = _make_sc_transpose(vocab, hid)
  table_rows = sc_t(jnp.transpose(table, (1, 0))).reshape(vocab, hid)

  sc_gather = _make_sc_gather(vocab, hid, n_rows, chunk=512)
  out_wide = sc_gather(table_rows, idx)
  embed = out_wide.reshape(b, s, 2 * hid)[:, :, :hid]

  w_full = jnp.transpose(conv_w, (2, 1, 0)).reshape(k * hid, hid)
  tc_conv = _make_tc_conv(b, s, hid, bs_blk=32)
  hidden = tc_conv(embed, w_full, conv_b.reshape(1, hid))

  return (embed, hidden)


# speedup vs baseline: 1.5436x; 1.5436x over previous
import functools

import jax
import jax.numpy as jnp
from jax import lax
from jax.experimental import pallas as pl
from jax.experimental.pallas import tpu as pltpu
from jax.experimental.pallas import tpu_sc as plsc


def _make_sc_transpose(vocab, hid):
  info = plsc.get_sparse_core_info()
  nc, ns = info.num_cores, info.num_subcores
  nw = nc * ns
  lanes = 2 * hid
  nb_full = vocab // lanes
  tail = vocab - nb_full * lanes
  max_iters = (nb_full + nw - 1) // nw

  mesh = plsc.VectorSubcoreMesh(core_axis_name="c", subcore_axis_name="s")

  @functools.partial(
      pl.kernel,
      mesh=mesh,
      compiler_params=pltpu.CompilerParams(needs_layout_passes=False),
      out_type=jax.ShapeDtypeStruct((vocab * hid,), jnp.float32),
      scratch_types=[
          pltpu.VMEM((hid, lanes), jnp.float32),
          pltpu.VMEM((hid, lanes), jnp.float32),
          pltpu.VMEM((hid * lanes,), jnp.float32),
          pltpu.VMEM((hid * lanes,), jnp.float32),
          pltpu.VMEM((hid, hid), jnp.float32),
          pltpu.SemaphoreType.DMA,
          pltpu.SemaphoreType.DMA,
          pltpu.SemaphoreType.DMA,
          pltpu.SemaphoreType.DMA,
          pltpu.SemaphoreType.DMA,
      ],
  )
  def sc_t(tin, out, vbuf0, vbuf1, tbuf0, tbuf1, vtail,
           in0, in1, out0, out1, tsem):
    wid = lax.axis_index("s") * nc + lax.axis_index("c")
    n_my = (nb_full - wid + nw - 1) // nw
    vbufs = (vbuf0, vbuf1)
    tbufs = (tbuf0, tbuf1)
    in_sems = (in0, in1)
    out_sems = (out0, out1)
    f16 = lax.iota(jnp.int32, 16)

    def start_in(i):
      g = wid + i * nw
      b = lax.rem(i, 2)
      for bb in range(2):
        @pl.when(b == bb)
        def _():
          pltpu.async_copy(tin.at[:, pl.ds(g * lanes, lanes)],
                           vbufs[bb], in_sems[bb])

    def transpose_block(bb, n_cols, src, dst):
      @plsc.parallel_loop(0, n_cols, step=1, unroll=8)
      def col(v):
        vv = jnp.zeros((16,), jnp.int32) + v
        for kk in range(hid // 16):
          vec = plsc.load_gather(src, [f16 + kk * 16, vv])
          dst[pl.ds(v * hid + kk * 16, 16)] = vec

    @pl.when(n_my > 0)
    def _():
      start_in(0)

      def body(i, carry):
        g = wid + i * nw
        b = lax.rem(i, 2)
        for bb in range(2):
          @pl.when(b == bb)
          def _():
            pltpu.make_async_copy(tin.at[:, pl.ds(g * lanes, lanes)],
                                  vbufs[bb], in_sems[bb]).wait()

        @pl.when(i + 1 < n_my)
        def _():
          start_in(i + 1)

        for bb in range(2):
          @pl.when(b == bb)
          def _():
            @pl.when(i >= 2)
            def _():
              pltpu.make_async_copy(
                  tbufs[bb], out.at[pl.ds(g * hid * lanes, hid * lanes)],
                  out_sems[bb]).wait()
            transpose_block(bb, lanes, vbufs[bb], tbufs[bb])
            pltpu.async_copy(
                tbufs[bb], out.at[pl.ds(g * hid * lanes, hid * lanes)],
                out_sems[bb])
        return carry

      lax.fori_loop(0, n_my, body, 0)

      def drain(i, carry):
        b = lax.rem(i, 2)
        g = wid + i * nw
        for bb in range(2):
          @pl.when((b == bb) & (i + 2 >= n_my))
          def _():
            pltpu.make_async_copy(
                tbufs[bb], out.at[pl.ds(g * hid * lanes, hid * lanes)],
                out_sems[bb]).wait()
        return carry
      lax.fori_loop(jnp.maximum(n_my - 2, 0), n_my, drain, 0)

    if tail:
      @pl.when(wid == nw - 1)
      def _():
        pltpu.async_copy(tin.at[:, pl.ds(nb_full * lanes, tail)], vtail, tsem)
        pltpu.make_async_copy(tin.at[:, pl.ds(nb_full * lanes, tail)], vtail,
                              tsem).wait()
        @plsc.parallel_loop(0, tail, step=1, unroll=8)
        def col(v):
          vv = jnp.zeros((16,), jnp.int32) + v
          for kk in range(hid // 16):
            vec = plsc.load_gather(vtail, [f16 + kk * 16, vv])
            tbuf0[pl.ds(v * hid + kk * 16, 16)] = vec
        pltpu.async_copy(
            tbuf0.at[pl.ds(0, tail * hid)],
            out.at[pl.ds(nb_full * lanes * hid, tail * hid)], tsem)
        pltpu.make_async_copy(
            tbuf0.at[pl.ds(0, tail * hid)],
            out.at[pl.ds(nb_full * lanes * hid, tail * hid)], tsem).wait()

  return sc_t


def _make_sc_gather(vocab, hid, n_rows, chunk):
  info = plsc.get_sparse_core_info()
  nc, ns = info.num_cores, info.num_subcores
  nw = nc * ns
  per_w = n_rows // nw
  assert n_rows % nw == 0 and per_w % chunk == 0
  n_chunks = per_w // chunk

  mesh = plsc.VectorSubcoreMesh(core_axis_name="c", subcore_axis_name="s")

  @functools.partial(
      pl.kernel,
      mesh=mesh,
      compiler_params=pltpu.CompilerParams(use_tc_tiling_on_sc=False),
      out_type=jax.ShapeDtypeStruct((n_rows, 2 * hid), jnp.float32),
      scratch_types=[
          pltpu.VMEM((chunk,), jnp.int32),
          pltpu.VMEM((chunk, hid), jnp.float32),
          pltpu.SemaphoreType.DMA,
      ],
  )
  def sc_gather(table_hbm, idx_hbm, out_hbm, idx_v, rows_v, sem):
    wid = lax.axis_index("s") * nc + lax.axis_index("c")
    w_base = wid * per_w

    def body(i, carry):
      base = w_base + i * chunk
      pltpu.sync_copy(idx_hbm.at[pl.ds(base, chunk)], idx_v)
      pltpu.async_copy(table_hbm.at[idx_v], rows_v, sem).wait()
      pltpu.sync_copy(rows_v, out_hbm.at[pl.ds(base, chunk), pl.ds(0, hid)])
      return carry

    lax.fori_loop(0, n_chunks, body, 0)

  return sc_gather


def _conv_body(x_ref, w_ref, b_ref, out_ref):
  bs, s, hid = x_ref.shape
  x = x_ref[...]
  zero = jnp.zeros((bs, 1, hid), jnp.float32)
  x_prev = jnp.concatenate([zero, x[:, :-1, :]], axis=1)
  x_next = jnp.concatenate([x[:, 1:, :], zero], axis=1)
  xcat = jnp.concatenate([x_prev, x, x_next], axis=2)
  y = jnp.dot(
      xcat.reshape(bs * s, 3 * hid), w_ref[...],
      preferred_element_type=jnp.float32)
  y = y.reshape(bs, s, hid)
  m = jnp.max(y, axis=1)
  out_ref[...] = jnp.maximum(m + b_ref[...], 0.0)


def _make_tc_conv(b, s, hid, bs_blk):
  assert b % bs_blk == 0
  grid = (b // bs_blk,)
  return pl.pallas_call(
      _conv_body,
      grid=grid,
      in_specs=[
          pl.BlockSpec((bs_blk, s, hid), lambda i: (i, 0, 0)),
          pl.BlockSpec((3 * hid, hid), lambda i: (0, 0)),
          pl.BlockSpec((1, hid), lambda i: (0, 0)),
      ],
      out_specs=pl.BlockSpec((bs_blk, hid), lambda i: (i, 0)),
      out_shape=jax.ShapeDtypeStruct((b, hid), jnp.float32),
  )


def kernel(input, table, conv_w, conv_b):
  b, s = input.shape
  vocab, hid = table.shape
  k = conv_w.shape[2]
  n_rows = b * s

  idx = input.reshape(n_rows)

  sc_t = _make_sc_transpose(vocab, hid)
  table_rows = sc_t(jnp.transpose(table, (1, 0))).reshape(vocab, hid)

  sc_gather = _make_sc_gather(vocab, hid, n_rows, chunk=512)
  out_wide = sc_gather(table_rows, idx)
  embed = out_wide.reshape(b, s, 2 * hid)[:, :, :hid]

  w_full = jnp.transpose(conv_w, (2, 1, 0)).reshape(k * hid, hid)
  tc_conv = _make_tc_conv(b, s, hid, bs_blk=32)
  hidden = tc_conv(embed, w_full, conv_b.reshape(1, hid))

  return (embed, hidden)


# speedup vs baseline: 2.0695x; 1.3407x over previous
import functools

import jax
import jax.numpy as jnp
from jax import lax
from jax.experimental import pallas as pl
from jax.experimental.pallas import tpu as pltpu
from jax.experimental.pallas import tpu_sc as plsc


def _make_sc_transpose(vocab, hid):
  info = plsc.get_sparse_core_info()
  nc, ns = info.num_cores, info.num_subcores
  nw = nc * ns
  lanes = 2 * hid
  nb_full = vocab // lanes
  tail = vocab - nb_full * lanes
  max_iters = (nb_full + nw - 1) // nw

  mesh = plsc.VectorSubcoreMesh(core_axis_name="c", subcore_axis_name="s")

  @functools.partial(
      pl.kernel,
      mesh=mesh,
      compiler_params=pltpu.CompilerParams(needs_layout_passes=False),
      out_type=jax.ShapeDtypeStruct((vocab * hid,), jnp.float32),
      scratch_types=[
          pltpu.VMEM((hid, lanes), jnp.float32),
          pltpu.VMEM((hid, lanes), jnp.float32),
          pltpu.VMEM((hid * lanes,), jnp.float32),
          pltpu.VMEM((hid * lanes,), jnp.float32),
          pltpu.VMEM((hid, hid), jnp.float32),
          pltpu.SemaphoreType.DMA,
          pltpu.SemaphoreType.DMA,
          pltpu.SemaphoreType.DMA,
          pltpu.SemaphoreType.DMA,
          pltpu.SemaphoreType.DMA,
      ],
  )
  def sc_t(tin, out, vbuf0, vbuf1, tbuf0, tbuf1, vtail,
           in0, in1, out0, out1, tsem):
    wid = lax.axis_index("s") * nc + lax.axis_index("c")
    n_my = (nb_full - wid + nw - 1) // nw
    vbufs = (vbuf0, vbuf1)
    tbufs = (tbuf0, tbuf1)
    in_sems = (in0, in1)
    out_sems = (out0, out1)
    f16 = lax.iota(jnp.int32, 16)

    def start_in(i):
      g = wid + i * nw
      b = lax.rem(i, 2)
      for bb in range(2):
        @pl.when(b == bb)
        def _():
          pltpu.async_copy(tin.at[:, pl.ds(g * lanes, lanes)],
                           vbufs[bb], in_sems[bb])

    def transpose_block(bb, n_cols, src, dst):
      @plsc.parallel_loop(0, n_cols, step=1, unroll=8)
      def col(v):
        vv = jnp.zeros((16,), jnp.int32) + v
        for kk in range(hid // 16):
          vec = plsc.load_gather(src, [f16 + kk * 16, vv])
          dst[pl.ds(v * hid + kk * 16, 16)] = vec

    @pl.when(n_my > 0)
    def _():
      start_in(0)

      def body(i, carry):
        g = wid + i * nw
        b = lax.rem(i, 2)
        for bb in range(2):
          @pl.when(b == bb)
          def _():
            pltpu.make_async_copy(tin.at[:, pl.ds(g * lanes, lanes)],
                                  vbufs[bb], in_sems[bb]).wait()

        @pl.when(i + 1 < n_my)
        def _():
          start_in(i + 1)

        for bb in range(2):
          @pl.when(b == bb)
          def _():
            @pl.when(i >= 2)
            def _():
              pltpu.make_async_copy(
                  tbufs[bb], out.at[pl.ds(g * hid * lanes, hid * lanes)],
                  out_sems[bb]).wait()
            transpose_block(bb, lanes, vbufs[bb], tbufs[bb])
            pltpu.async_copy(
                tbufs[bb], out.at[pl.ds(g * hid * lanes, hid * lanes)],
                out_sems[bb])
        return carry

      lax.fori_loop(0, n_my, body, 0)

      def drain(i, carry):
        b = lax.rem(i, 2)
        g = wid + i * nw
        for bb in range(2):
          @pl.when((b == bb) & (i + 2 >= n_my))
          def _():
            pltpu.make_async_copy(
                tbufs[bb], out.at[pl.ds(g * hid * lanes, hid * lanes)],
                out_sems[bb]).wait()
        return carry
      lax.fori_loop(jnp.maximum(n_my - 2, 0), n_my, drain, 0)

    if tail:
      @pl.when(wid == nw - 1)
      def _():
        pltpu.async_copy(tin.at[:, pl.ds(nb_full * lanes, tail)], vtail, tsem)
        pltpu.make_async_copy(tin.at[:, pl.ds(nb_full * lanes, tail)], vtail,
                              tsem).wait()
        @plsc.parallel_loop(0, tail, step=1, unroll=8)
        def col(v):
          vv = jnp.zeros((16,), jnp.int32) + v
          for kk in range(hid // 16):
            vec = plsc.load_gather(vtail, [f16 + kk * 16, vv])
            tbuf0[pl.ds(v * hid + kk * 16, 16)] = vec
        pltpu.async_copy(
            tbuf0.at[pl.ds(0, tail * hid)],
            out.at[pl.ds(nb_full * lanes * hid, tail * hid)], tsem)
        pltpu.make_async_copy(
            tbuf0.at[pl.ds(0, tail * hid)],
            out.at[pl.ds(nb_full * lanes * hid, tail * hid)], tsem).wait()

  return sc_t


def _tt_body(x_ref, out_ref):
  hid, vb = x_ref.shape
  y = jnp.transpose(x_ref[...], (1, 0))
  out_ref[...] = jnp.concatenate([y[:vb // 2, :], y[vb // 2:, :]], axis=1)


def _make_tc_transpose(vocab, hid, vb):
  n_blk = (vocab + vb - 1) // vb
  return pl.pallas_call(
      _tt_body,
      grid=(n_blk,),
      in_specs=[pl.BlockSpec((hid, vb), lambda i: (0, i))],
      out_specs=pl.BlockSpec((vb // 2, 2 * hid), lambda i: (i, 0)),
      out_shape=jax.ShapeDtypeStruct((n_blk * vb // 2, 2 * hid), jnp.float32),
  )


def _make_sc_gather(vocab, hid, n_rows, chunk):
  info = plsc.get_sparse_core_info()
  nc, ns = info.num_cores, info.num_subcores
  nw = nc * ns
  per_w = n_rows // nw
  assert n_rows % nw == 0 and per_w % chunk == 0
  n_chunks = per_w // chunk

  mesh = plsc.VectorSubcoreMesh(core_axis_name="c", subcore_axis_name="s")

  @functools.partial(
      pl.kernel,
      mesh=mesh,
      compiler_params=pltpu.CompilerParams(use_tc_tiling_on_sc=False),
      out_type=jax.ShapeDtypeStruct((n_rows, 2 * hid), jnp.float32),
      scratch_types=[
          pltpu.VMEM((chunk,), jnp.int32),
          pltpu.VMEM((chunk, hid), jnp.float32),
          pltpu.SemaphoreType.DMA,
      ],
  )
  def sc_gather(table_hbm, idx_hbm, out_hbm, idx_v, rows_v, sem):
    wid = lax.axis_index("s") * nc + lax.axis_index("c")
    w_base = wid * per_w

    def body(i, carry):
      base = w_base + i * chunk
      pltpu.sync_copy(idx_hbm.at[pl.ds(base, chunk)], idx_v)
      pltpu.async_copy(table_hbm.at[idx_v], rows_v, sem).wait()
      pltpu.sync_copy(rows_v, out_hbm.at[pl.ds(base, chunk), pl.ds(0, hid)])
      return carry

    lax.fori_loop(0, n_chunks, body, 0)

  return sc_gather


def _conv_body(x_ref, w_ref, b_ref, out_ref):
  bs, s, hid = x_ref.shape
  x = x_ref[...]
  zero = jnp.zeros((bs, 1, hid), jnp.float32)
  x_prev = jnp.concatenate([zero, x[:, :-1, :]], axis=1)
  x_next = jnp.concatenate([x[:, 1:, :], zero], axis=1)
  xcat = jnp.concatenate([x_prev, x, x_next], axis=2)
  y = jnp.dot(
      xcat.reshape(bs * s, 3 * hid), w_ref[...],
      preferred_element_type=jnp.float32)
  y = y.reshape(bs, s, hid)
  m = jnp.max(y, axis=1)
  out_ref[...] = jnp.maximum(m + b_ref[...], 0.0)


def _make_tc_conv(b, s, hid, bs_blk):
  assert b % bs_blk == 0
  grid = (b // bs_blk,)
  return pl.pallas_call(
      _conv_body,
      grid=grid,
      in_specs=[
          pl.BlockSpec((bs_blk, s, hid), lambda i: (i, 0, 0)),
          pl.BlockSpec((3 * hid, hid), lambda i: (0, 0)),
          pl.BlockSpec((1, hid), lambda i: (0, 0)),
      ],
      out_specs=pl.BlockSpec((bs_blk, hid), lambda i: (i, 0)),
      out_shape=jax.ShapeDtypeStruct((b, hid), jnp.float32),
  )


def kernel(input, table, conv_w, conv_b):
  b, s = input.shape
  vocab, hid = table.shape
  k = conv_w.shape[2]
  n_rows = b * s

  idx = input.reshape(n_rows)

  vb = 2048
  tc_t = _make_tc_transpose(vocab, hid, vb=vb)
  packed = tc_t(jnp.transpose(table, (1, 0)))
  table_rows = packed.reshape(packed.shape[0] * 2, hid)

  j = idx % vb
  idx2 = (idx // vb) * vb + (j % (vb // 2)) * 2 + j // (vb // 2)

  sc_gather = _make_sc_gather(vocab, hid, n_rows, chunk=512)
  out_wide = sc_gather(table_rows, idx2)
  embed = out_wide.reshape(b, s, 2 * hid)[:, :, :hid]

  w_full = jnp.transpose(conv_w, (2, 1, 0)).reshape(k * hid, hid)
  tc_conv = _make_tc_conv(b, s, hid, bs_blk=32)
  hidden = tc_conv(embed, w_full, conv_b.reshape(1, hid))

  return (embed, hidden)


# speedup vs baseline: 2.4053x; 1.1622x over previous
import functools

import jax
import jax.numpy as jnp
from jax import lax
from jax.experimental import pallas as pl
from jax.experimental.pallas import tpu as pltpu
from jax.experimental.pallas import tpu_sc as plsc


def _make_sc_transpose(vocab, hid):
  info = plsc.get_sparse_core_info()
  nc, ns = info.num_cores, info.num_subcores
  nw = nc * ns
  lanes = 2 * hid
  nb_full = vocab // lanes
  tail = vocab - nb_full * lanes
  max_iters = (nb_full + nw - 1) // nw

  mesh = plsc.VectorSubcoreMesh(core_axis_name="c", subcore_axis_name="s")

  @functools.partial(
      pl.kernel,
      mesh=mesh,
      compiler_params=pltpu.CompilerParams(needs_layout_passes=False),
      out_type=jax.ShapeDtypeStruct((vocab * hid,), jnp.float32),
      scratch_types=[
          pltpu.VMEM((hid, lanes), jnp.float32),
          pltpu.VMEM((hid, lanes), jnp.float32),
          pltpu.VMEM((hid * lanes,), jnp.float32),
          pltpu.VMEM((hid * lanes,), jnp.float32),
          pltpu.VMEM((hid, hid), jnp.float32),
          pltpu.SemaphoreType.DMA,
          pltpu.SemaphoreType.DMA,
          pltpu.SemaphoreType.DMA,
          pltpu.SemaphoreType.DMA,
          pltpu.SemaphoreType.DMA,
      ],
  )
  def sc_t(tin, out, vbuf0, vbuf1, tbuf0, tbuf1, vtail,
           in0, in1, out0, out1, tsem):
    wid = lax.axis_index("s") * nc + lax.axis_index("c")
    n_my = (nb_full - wid + nw - 1) // nw
    vbufs = (vbuf0, vbuf1)
    tbufs = (tbuf0, tbuf1)
    in_sems = (in0, in1)
    out_sems = (out0, out1)
    f16 = lax.iota(jnp.int32, 16)

    def start_in(i):
      g = wid + i * nw
      b = lax.rem(i, 2)
      for bb in range(2):
        @pl.when(b == bb)
        def _():
          pltpu.async_copy(tin.at[:, pl.ds(g * lanes, lanes)],
                           vbufs[bb], in_sems[bb])

    def transpose_block(bb, n_cols, src, dst):
      @plsc.parallel_loop(0, n_cols, step=1, unroll=8)
      def col(v):
        vv = jnp.zeros((16,), jnp.int32) + v
        for kk in range(hid // 16):
          vec = plsc.load_gather(src, [f16 + kk * 16, vv])
          dst[pl.ds(v * hid + kk * 16, 16)] = vec

    @pl.when(n_my > 0)
    def _():
      start_in(0)

      def body(i, carry):
        g = wid + i * nw
        b = lax.rem(i, 2)
        for bb in range(2):
          @pl.when(b == bb)
          def _():
            pltpu.make_async_copy(tin.at[:, pl.ds(g * lanes, lanes)],
                                  vbufs[bb], in_sems[bb]).wait()

        @pl.when(i + 1 < n_my)
        def _():
          start_in(i + 1)

        for bb in range(2):
          @pl.when(b == bb)
          def _():
            @pl.when(i >= 2)
            def _():
              pltpu.make_async_copy(
                  tbufs[bb], out.at[pl.ds(g * hid * lanes, hid * lanes)],
                  out_sems[bb]).wait()
            transpose_block(bb, lanes, vbufs[bb], tbufs[bb])
            pltpu.async_copy(
                tbufs[bb], out.at[pl.ds(g * hid * lanes, hid * lanes)],
                out_sems[bb])
        return carry

      lax.fori_loop(0, n_my, body, 0)

      def drain(i, carry):
        b = lax.rem(i, 2)
        g = wid + i * nw
        for bb in range(2):
          @pl.when((b == bb) & (i + 2 >= n_my))
          def _():
            pltpu.make_async_copy(
                tbufs[bb], out.at[pl.ds(g * hid * lanes, hid * lanes)],
                out_sems[bb]).wait()
        return carry
      lax.fori_loop(jnp.maximum(n_my - 2, 0), n_my, drain, 0)

    if tail:
      @pl.when(wid == nw - 1)
      def _():
        pltpu.async_copy(tin.at[:, pl.ds(nb_full * lanes, tail)], vtail, tsem)
        pltpu.make_async_copy(tin.at[:, pl.ds(nb_full * lanes, tail)], vtail,
                              tsem).wait()
        @plsc.parallel_loop(0, tail, step=1, unroll=8)
        def col(v):
          vv = jnp.zeros((16,), jnp.int32) + v
          for kk in range(hid // 16):
            vec = plsc.load_gather(vtail, [f16 + kk * 16, vv])
            tbuf0[pl.ds(v * hid + kk * 16, 16)] = vec
        pltpu.async_copy(
            tbuf0.at[pl.ds(0, tail * hid)],
            out.at[pl.ds(nb_full * lanes * hid, tail * hid)], tsem)
        pltpu.make_async_copy(
            tbuf0.at[pl.ds(0, tail * hid)],
            out.at[pl.ds(nb_full * lanes * hid, tail * hid)], tsem).wait()

  return sc_t


def _tt_body(x_ref, out_ref):
  hid, vb = x_ref.shape
  y = jnp.transpose(x_ref[...], (1, 0))
  out_ref[...] = jnp.concatenate([y[:vb // 2, :], y[vb // 2:, :]], axis=1)


def _make_tc_transpose(vocab, hid, vb):
  n_blk = (vocab + vb - 1) // vb
  return pl.pallas_call(
      _tt_body,
      grid=(n_blk,),
      in_specs=[pl.BlockSpec((hid, vb), lambda i: (0, i))],
      out_specs=pl.BlockSpec((vb // 2, 2 * hid), lambda i: (i, 0)),
      out_shape=jax.ShapeDtypeStruct((n_blk * vb // 2, 2 * hid), jnp.float32),
  )


def _make_sc_gather(vocab, hid, n_rows, chunk):
  info = plsc.get_sparse_core_info()
  nc, ns = info.num_cores, info.num_subcores
  nw = nc * ns
  per_w = n_rows // nw
  assert n_rows % nw == 0 and per_w % chunk == 0
  n_chunks = per_w // chunk

  mesh = plsc.VectorSubcoreMesh(core_axis_name="c", subcore_axis_name="s")

  @functools.partial(
      pl.kernel,
      mesh=mesh,
      compiler_params=pltpu.CompilerParams(use_tc_tiling_on_sc=False),
      out_type=jax.ShapeDtypeStruct((n_rows, 2 * hid), jnp.float32),
      scratch_types=[
          pltpu.VMEM((chunk,), jnp.int32),
          pltpu.VMEM((chunk, hid), jnp.float32),
          pltpu.SemaphoreType.DMA,
      ],
  )
  def sc_gather(table_hbm, idx_hbm, out_hbm, idx_v, rows_v, sem):
    wid = lax.axis_index("s") * nc + lax.axis_index("c")
    w_base = wid * per_w

    def body(i, carry):
      base = w_base + i * chunk
      pltpu.sync_copy(idx_hbm.at[pl.ds(base, chunk)], idx_v)
      pltpu.async_copy(table_hbm.at[idx_v], rows_v, sem).wait()
      pltpu.sync_copy(rows_v, out_hbm.at[pl.ds(base, chunk), pl.ds(0, hid)])
      return carry

    lax.fori_loop(0, n_chunks, body, 0)

  return sc_gather


def _conv_body(x_ref, w_ref, b_ref, out_ref):
  bs, s, hid = x_ref.shape
  x = x_ref[...]
  zero = jnp.zeros((bs, 1, hid), jnp.float32)
  x_prev = jnp.concatenate([zero, x[:, :-1, :]], axis=1)
  x_next = jnp.concatenate([x[:, 1:, :], zero], axis=1)
  xcat = jnp.concatenate([x_prev, x, x_next], axis=2)
  y = jnp.dot(
      xcat.reshape(bs * s, 3 * hid), w_ref[...],
      preferred_element_type=jnp.float32)
  y = y.reshape(bs, s, hid)
  m = jnp.max(y, axis=1)
  out_ref[...] = jnp.maximum(m + b_ref[...], 0.0)


def _make_tc_conv(b, s, hid, bs_blk):
  assert b % bs_blk == 0
  grid = (b // bs_blk,)
  return pl.pallas_call(
      _conv_body,
      grid=grid,
      in_specs=[
          pl.BlockSpec((bs_blk, s, hid), lambda i: (i, 0, 0)),
          pl.BlockSpec((3 * hid, hid), lambda i: (0, 0)),
          pl.BlockSpec((1, hid), lambda i: (0, 0)),
      ],
      out_specs=pl.BlockSpec((bs_blk, hid), lambda i: (i, 0)),
      out_shape=jax.ShapeDtypeStruct((b, hid), jnp.float32),
  )


def kernel(input, table, conv_w, conv_b):
  b, s = input.shape
  vocab, hid = table.shape
  k = conv_w.shape[2]
  n_rows = b * s

  idx = input.reshape(n_rows)

  vb = 4096
  tc_t = _make_tc_transpose(vocab, hid, vb=vb)
  packed = tc_t(jnp.transpose(table, (1, 0)))
  table_rows = packed.reshape(packed.shape[0] * 2, hid)

  j = idx % vb
  idx2 = (idx // vb) * vb + (j % (vb // 2)) * 2 + j // (vb // 2)

  sc_gather = _make_sc_gather(vocab, hid, n_rows, chunk=512)
  out_wide = sc_gather(table_rows, idx2)
  embed = out_wide.reshape(b, s, 2 * hid)[:, :, :hid]

  w_full = jnp.transpose(conv_w, (2, 1, 0)).reshape(k * hid, hid)
  tc_conv = _make_tc_conv(b, s, hid, bs_blk=64)
  hidden = tc_conv(embed, w_full, conv_b.reshape(1, hid))

  return (embed, hidden)


# speedup vs baseline: 2.6135x; 1.0866x over previous
import functools

import jax
import jax.numpy as jnp
from jax import lax
from jax.experimental import pallas as pl
from jax.experimental.pallas import tpu as pltpu
from jax.experimental.pallas import tpu_sc as plsc


def _make_sc_transpose(vocab, hid):
  info = plsc.get_sparse_core_info()
  nc, ns = info.num_cores, info.num_subcores
  nw = nc * ns
  lanes = 2 * hid
  nb_full = vocab // lanes
  tail = vocab - nb_full * lanes
  max_iters = (nb_full + nw - 1) // nw

  mesh = plsc.VectorSubcoreMesh(core_axis_name="c", subcore_axis_name="s")

  @functools.partial(
      pl.kernel,
      mesh=mesh,
      compiler_params=pltpu.CompilerParams(needs_layout_passes=False),
      out_type=jax.ShapeDtypeStruct((vocab * hid,), jnp.float32),
      scratch_types=[
          pltpu.VMEM((hid, lanes), jnp.float32),
          pltpu.VMEM((hid, lanes), jnp.float32),
          pltpu.VMEM((hid * lanes,), jnp.float32),
          pltpu.VMEM((hid * lanes,), jnp.float32),
          pltpu.VMEM((hid, hid), jnp.float32),
          pltpu.SemaphoreType.DMA,
          pltpu.SemaphoreType.DMA,
          pltpu.SemaphoreType.DMA,
          pltpu.SemaphoreType.DMA,
          pltpu.SemaphoreType.DMA,
      ],
  )
  def sc_t(tin, out, vbuf0, vbuf1, tbuf0, tbuf1, vtail,
           in0, in1, out0, out1, tsem):
    wid = lax.axis_index("s") * nc + lax.axis_index("c")
    n_my = (nb_full - wid + nw - 1) // nw
    vbufs = (vbuf0, vbuf1)
    tbufs = (tbuf0, tbuf1)
    in_sems = (in0, in1)
    out_sems = (out0, out1)
    f16 = lax.iota(jnp.int32, 16)

    def start_in(i):
      g = wid + i * nw
      b = lax.rem(i, 2)
      for bb in range(2):
        @pl.when(b == bb)
        def _():
          pltpu.async_copy(tin.at[:, pl.ds(g * lanes, lanes)],
                           vbufs[bb], in_sems[bb])

    def transpose_block(bb, n_cols, src, dst):
      @plsc.parallel_loop(0, n_cols, step=1, unroll=8)
      def col(v):
        vv = jnp.zeros((16,), jnp.int32) + v
        for kk in range(hid // 16):
          vec = plsc.load_gather(src, [f16 + kk * 16, vv])
          dst[pl.ds(v * hid + kk * 16, 16)] = vec

    @pl.when(n_my > 0)
    def _():
      start_in(0)

      def body(i, carry):
        g = wid + i * nw
        b = lax.rem(i, 2)
        for bb in range(2):
          @pl.when(b == bb)
          def _():
            pltpu.make_async_copy(tin.at[:, pl.ds(g * lanes, lanes)],
                                  vbufs[bb], in_sems[bb]).wait()

        @pl.when(i + 1 < n_my)
        def _():
          start_in(i + 1)

        for bb in range(2):
          @pl.when(b == bb)
          def _():
            @pl.when(i >= 2)
            def _():
              pltpu.make_async_copy(
                  tbufs[bb], out.at[pl.ds(g * hid * lanes, hid * lanes)],
                  out_sems[bb]).wait()
            transpose_block(bb, lanes, vbufs[bb], tbufs[bb])
            pltpu.async_copy(
                tbufs[bb], out.at[pl.ds(g * hid * lanes, hid * lanes)],
                out_sems[bb])
        return carry

      lax.fori_loop(0, n_my, body, 0)

      def drain(i, carry):
        b = lax.rem(i, 2)
        g = wid + i * nw
        for bb in range(2):
          @pl.when((b == bb) & (i + 2 >= n_my))
          def _():
            pltpu.make_async_copy(
                tbufs[bb], out.at[pl.ds(g * hid * lanes, hid * lanes)],
                out_sems[bb]).wait()
        return carry
      lax.fori_loop(jnp.maximum(n_my - 2, 0), n_my, drain, 0)

    if tail:
      @pl.when(wid == nw - 1)
      def _():
        pltpu.async_copy(tin.at[:, pl.ds(nb_full * lanes, tail)], vtail, tsem)
        pltpu.make_async_copy(tin.at[:, pl.ds(nb_full * lanes, tail)], vtail,
                              tsem).wait()
        @plsc.parallel_loop(0, tail, step=1, unroll=8)
        def col(v):
          vv = jnp.zeros((16,), jnp.int32) + v
          for kk in range(hid // 16):
            vec = plsc.load_gather(vtail, [f16 + kk * 16, vv])
            tbuf0[pl.ds(v * hid + kk * 16, 16)] = vec
        pltpu.async_copy(
            tbuf0.at[pl.ds(0, tail * hid)],
            out.at[pl.ds(nb_full * lanes * hid, tail * hid)], tsem)
        pltpu.make_async_copy(
            tbuf0.at[pl.ds(0, tail * hid)],
            out.at[pl.ds(nb_full * lanes * hid, tail * hid)], tsem).wait()

  return sc_t


def _tt_body(x_ref, out_ref):
  hid, vb = x_ref.shape
  y = jnp.transpose(x_ref[...], (1, 0))
  out_ref[...] = jnp.concatenate([y[:vb // 2, :], y[vb // 2:, :]], axis=1)


def _make_tc_transpose(vocab, hid, vb):
  n_blk = (vocab + vb - 1) // vb
  return pl.pallas_call(
      _tt_body,
      grid=(n_blk,),
      in_specs=[pl.BlockSpec((hid, vb), lambda i: (0, i))],
      out_specs=pl.BlockSpec((vb // 2, 2 * hid), lambda i: (i, 0)),
      out_shape=jax.ShapeDtypeStruct((n_blk * vb // 2, 2 * hid), jnp.float32),
  )


def _make_sc_gather(vocab, hid, n_rows, chunk):
  info = plsc.get_sparse_core_info()
  nc, ns = info.num_cores, info.num_subcores
  nw = nc * ns
  per_w = n_rows // nw
  assert n_rows % nw == 0 and per_w % chunk == 0
  n_chunks = per_w // chunk

  mesh = plsc.VectorSubcoreMesh(core_axis_name="c", subcore_axis_name="s")

  @functools.partial(
      pl.kernel,
      mesh=mesh,
      compiler_params=pltpu.CompilerParams(use_tc_tiling_on_sc=False),
      out_type=jax.ShapeDtypeStruct((n_rows, 2 * hid), jnp.float32),
      scratch_types=[
          pltpu.VMEM((chunk,), jnp.int32),
          pltpu.VMEM((chunk, hid), jnp.float32),
          pltpu.SemaphoreType.DMA,
      ],
  )
  def sc_gather(table_hbm, idx_hbm, out_hbm, idx_v, rows_v, sem):
    wid = lax.axis_index("s") * nc + lax.axis_index("c")
    w_base = wid * per_w

    def body(i, carry):
      base = w_base + i * chunk
      pltpu.sync_copy(idx_hbm.at[pl.ds(base, chunk)], idx_v)
      pltpu.async_copy(table_hbm.at[idx_v], rows_v, sem).wait()
      pltpu.sync_copy(rows_v, out_hbm.at[pl.ds(base, chunk), pl.ds(0, hid)])
      return carry

    lax.fori_loop(0, n_chunks, body, 0)

  return sc_gather


def _conv_body(x_ref, w_ref, b_ref, out_ref):
  bs, s, hid = x_ref.shape
  x = x_ref[...]
  zero = jnp.zeros((bs, 1, hid), jnp.float32)
  x_prev = jnp.concatenate([zero, x[:, :-1, :]], axis=1)
  x_next = jnp.concatenate([x[:, 1:, :], zero], axis=1)
  xcat = jnp.concatenate([x_prev, x, x_next], axis=2)
  y = jnp.dot(
      xcat.reshape(bs * s, 3 * hid), w_ref[...],
      preferred_element_type=jnp.float32)
  y = y.reshape(bs, s, hid)
  m = jnp.max(y, axis=1)
  out_ref[...] = jnp.maximum(m + b_ref[...], 0.0)


def _make_tc_conv(b, s, hid, bs_blk):
  assert b % bs_blk == 0
  grid = (b // bs_blk,)
  return pl.pallas_call(
      _conv_body,
      grid=grid,
      in_specs=[
          pl.BlockSpec((bs_blk, s, hid), lambda i: (i, 0, 0)),
          pl.BlockSpec((3 * hid, hid), lambda i: (0, 0)),
          pl.BlockSpec((1, hid), lambda i: (0, 0)),
      ],
      out_specs=pl.BlockSpec((bs_blk, hid), lambda i: (i, 0)),
      out_shape=jax.ShapeDtypeStruct((b, hid), jnp.float32),
  )


def kernel(input, table, conv_w, conv_b):
  b, s = input.shape
  vocab, hid = table.shape
  k = conv_w.shape[2]
  n_rows = b * s

  idx = input.reshape(n_rows)

  vb = 8192
  tc_t = _make_tc_transpose(vocab, hid, vb=vb)
  packed = tc_t(jnp.transpose(table, (1, 0)))
  table_rows = packed.reshape(packed.shape[0] * 2, hid)

  j = idx % vb
  idx2 = (idx // vb) * vb + (j % (vb // 2)) * 2 + j // (vb // 2)

  sc_gather = _make_sc_gather(vocab, hid, n_rows, chunk=512)
  out_wide = sc_gather(table_rows, idx2)
  embed = out_wide.reshape(b, s, 2 * hid)[:, :, :hid]

  w_full = jnp.transpose(conv_w, (2, 1, 0)).reshape(k * hid, hid)
  tc_conv = _make_tc_conv(b, s, hid, bs_blk=128)
  hidden = tc_conv(embed, w_full, conv_b.reshape(1, hid))

  return (embed, hidden)


# speedup vs baseline: 2.8709x; 1.0985x over previous
import functools

import jax
import jax.numpy as jnp
from jax import lax
from jax.experimental import pallas as pl
from jax.experimental.pallas import tpu as pltpu
from jax.experimental.pallas import tpu_sc as plsc


def _make_sc_transpose(vocab, hid):
  info = plsc.get_sparse_core_info()
  nc, ns = info.num_cores, info.num_subcores
  nw = nc * ns
  lanes = 2 * hid
  nb_full = vocab // lanes
  tail = vocab - nb_full * lanes
  max_iters = (nb_full + nw - 1) // nw

  mesh = plsc.VectorSubcoreMesh(core_axis_name="c", subcore_axis_name="s")

  @functools.partial(
      pl.kernel,
      mesh=mesh,
      compiler_params=pltpu.CompilerParams(needs_layout_passes=False),
      out_type=jax.ShapeDtypeStruct((vocab * hid,), jnp.float32),
      scratch_types=[
          pltpu.VMEM((hid, lanes), jnp.float32),
          pltpu.VMEM((hid, lanes), jnp.float32),
          pltpu.VMEM((hid * lanes,), jnp.float32),
          pltpu.VMEM((hid * lanes,), jnp.float32),
          pltpu.VMEM((hid, hid), jnp.float32),
          pltpu.SemaphoreType.DMA,
          pltpu.SemaphoreType.DMA,
          pltpu.SemaphoreType.DMA,
          pltpu.SemaphoreType.DMA,
          pltpu.SemaphoreType.DMA,
      ],
  )
  def sc_t(tin, out, vbuf0, vbuf1, tbuf0, tbuf1, vtail,
           in0, in1, out0, out1, tsem):
    wid = lax.axis_index("s") * nc + lax.axis_index("c")
    n_my = (nb_full - wid + nw - 1) // nw
    vbufs = (vbuf0, vbuf1)
    tbufs = (tbuf0, tbuf1)
    in_sems = (in0, in1)
    out_sems = (out0, out1)
    f16 = lax.iota(jnp.int32, 16)

    def start_in(i):
      g = wid + i * nw
      b = lax.rem(i, 2)
      for bb in range(2):
        @pl.when(b == bb)
        def _():
          pltpu.async_copy(tin.at[:, pl.ds(g * lanes, lanes)],
                           vbufs[bb], in_sems[bb])

    def transpose_block(bb, n_cols, src, dst):
      @plsc.parallel_loop(0, n_cols, step=1, unroll=8)
      def col(v):
        vv = jnp.zeros((16,), jnp.int32) + v
        for kk in range(hid // 16):
          vec = plsc.load_gather(src, [f16 + kk * 16, vv])
          dst[pl.ds(v * hid + kk * 16, 16)] = vec

    @pl.when(n_my > 0)
    def _():
      start_in(0)

      def body(i, carry):
        g = wid + i * nw
        b = lax.rem(i, 2)
        for bb in range(2):
          @pl.when(b == bb)
          def _():
            pltpu.make_async_copy(tin.at[:, pl.ds(g * lanes, lanes)],
                                  vbufs[bb], in_sems[bb]).wait()

        @pl.when(i + 1 < n_my)
        def _():
          start_in(i + 1)

        for bb in range(2):
          @pl.when(b == bb)
          def _():
            @pl.when(i >= 2)
            def _():
              pltpu.make_async_copy(
                  tbufs[bb], out.at[pl.ds(g * hid * lanes, hid * lanes)],
                  out_sems[bb]).wait()
            transpose_block(bb, lanes, vbufs[bb], tbufs[bb])
            pltpu.async_copy(
                tbufs[bb], out.at[pl.ds(g * hid * lanes, hid * lanes)],
                out_sems[bb])
        return carry

      lax.fori_loop(0, n_my, body, 0)

      def drain(i, carry):
        b = lax.rem(i, 2)
        g = wid + i * nw
        for bb in range(2):
          @pl.when((b == bb) & (i + 2 >= n_my))
          def _():
            pltpu.make_async_copy(
                tbufs[bb], out.at[pl.ds(g * hid * lanes, hid * lanes)],
                out_sems[bb]).wait()
        return carry
      lax.fori_loop(jnp.maximum(n_my - 2, 0), n_my, drain, 0)

    if tail:
      @pl.when(wid == nw - 1)
      def _():
        pltpu.async_copy(tin.at[:, pl.ds(nb_full * lanes, tail)], vtail, tsem)
        pltpu.make_async_copy(tin.at[:, pl.ds(nb_full * lanes, tail)], vtail,
                              tsem).wait()
        @plsc.parallel_loop(0, tail, step=1, unroll=8)
        def col(v):
          vv = jnp.zeros((16,), jnp.int32) + v
          for kk in range(hid // 16):
            vec = plsc.load_gather(vtail, [f16 + kk * 16, vv])
            tbuf0[pl.ds(v * hid + kk * 16, 16)] = vec
        pltpu.async_copy(
            tbuf0.at[pl.ds(0, tail * hid)],
            out.at[pl.ds(nb_full * lanes * hid, tail * hid)], tsem)
        pltpu.make_async_copy(
            tbuf0.at[pl.ds(0, tail * hid)],
            out.at[pl.ds(nb_full * lanes * hid, tail * hid)], tsem).wait()

  return sc_t


def _tt_body(x_ref, out_ref):
  hid, vb = x_ref.shape
  y = jnp.transpose(x_ref[...], (1, 0))
  out_ref[...] = jnp.concatenate([y[:vb // 2, :], y[vb // 2:, :]], axis=1)


def _make_tc_transpose(vocab, hid, vb):
  n_blk = (vocab + vb - 1) // vb
  return pl.pallas_call(
      _tt_body,
      grid=(n_blk,),
      in_specs=[pl.BlockSpec((hid, vb), lambda i: (0, i))],
      out_specs=pl.BlockSpec((vb // 2, 2 * hid), lambda i: (i, 0)),
      out_shape=jax.ShapeDtypeStruct((n_blk * vb // 2, 2 * hid), jnp.float32),
  )


def _make_sc_gather(vocab, hid, n_rows, chunk):
  info = plsc.get_sparse_core_info()
  nc, ns = info.num_cores, info.num_subcores
  nw = nc * ns
  per_w = n_rows // nw
  assert n_rows % nw == 0 and per_w % chunk == 0
  n_chunks = per_w // chunk

  mesh = plsc.VectorSubcoreMesh(core_axis_name="c", subcore_axis_name="s")

  @functools.partial(
      pl.kernel,
      mesh=mesh,
      compiler_params=pltpu.CompilerParams(use_tc_tiling_on_sc=False),
      out_type=jax.ShapeDtypeStruct((n_rows, 2 * hid), jnp.float32),
      scratch_types=[
          pltpu.VMEM((chunk,), jnp.int32),
          pltpu.VMEM((chunk, hid), jnp.float32),
          pltpu.SemaphoreType.DMA,
      ],
  )
  def sc_gather(table_hbm, idx_hbm, out_hbm, idx_v, rows_v, sem):
    wid = lax.axis_index("s") * nc + lax.axis_index("c")
    w_base = wid * per_w

    def body(i, carry):
      base = w_base + i * chunk
      pltpu.sync_copy(idx_hbm.at[pl.ds(base, chunk)], idx_v)
      pltpu.async_copy(table_hbm.at[idx_v], rows_v, sem).wait()
      pltpu.sync_copy(rows_v, out_hbm.at[pl.ds(base, chunk), pl.ds(0, hid)])
      return carry

    lax.fori_loop(0, n_chunks, body, 0)

  return sc_gather


def _conv_body(x_ref, w_ref, b_ref, out_ref):
  bs, s, hid = x_ref.shape
  x = x_ref[...]
  zero = jnp.zeros((bs, 1, hid), jnp.float32)
  x_prev = jnp.concatenate([zero, x[:, :-1, :]], axis=1)
  x_next = jnp.concatenate([x[:, 1:, :], zero], axis=1)
  xcat = jnp.concatenate([x_prev, x, x_next], axis=2)
  y = jnp.dot(
      xcat.reshape(bs * s, 3 * hid), w_ref[...],
      preferred_element_type=jnp.float32)
  y = y.reshape(bs, s, hid)
  m = jnp.max(y, axis=1)
  out_ref[...] = jnp.maximum(m + b_ref[...], 0.0)


def _make_tc_conv(b, s, hid, bs_blk):
  assert b % bs_blk == 0
  grid = (b // bs_blk,)
  return pl.pallas_call(
      _conv_body,
      grid=grid,
      in_specs=[
          pl.BlockSpec((bs_blk, s, hid), lambda i: (i, 0, 0)),
          pl.BlockSpec((3 * hid, hid), lambda i: (0, 0)),
          pl.BlockSpec((1, hid), lambda i: (0, 0)),
      ],
      out_specs=pl.BlockSpec((bs_blk, hid), lambda i: (i, 0)),
      out_shape=jax.ShapeDtypeStruct((b, hid), jnp.float32),
  )


def kernel(input, table, conv_w, conv_b):
  b, s = input.shape
  vocab, hid = table.shape
  k = conv_w.shape[2]
  n_rows = b * s

  idx = input.reshape(n_rows)

  vb = 16384
  tc_t = _make_tc_transpose(vocab, hid, vb=vb)
  packed = tc_t(jnp.transpose(table, (1, 0)))
  table_rows = packed.reshape(packed.shape[0] * 2, hid)

  j = idx % vb
  idx2 = (idx // vb) * vb + (j % (vb // 2)) * 2 + j // (vb // 2)

  sc_gather = _make_sc_gather(vocab, hid, n_rows, chunk=1024)
  out_wide = sc_gather(table_rows, idx2)
  embed = out_wide.reshape(b, s, 2 * hid)[:, :, :hid]

  w_full = jnp.transpose(conv_w, (2, 1, 0)).reshape(k * hid, hid)
  tc_conv = _make_tc_conv(b, s, hid, bs_blk=128)
  hidden = tc_conv(embed, w_full, conv_b.reshape(1, hid))

  return (embed, hidden)


# speedup vs baseline: 2.9588x; 1.0306x over previous
import functools

import jax
import jax.numpy as jnp
from jax import lax
from jax.experimental import pallas as pl
from jax.experimental.pallas import tpu as pltpu
from jax.experimental.pallas import tpu_sc as plsc


def _make_sc_transpose(vocab, hid):
  info = plsc.get_sparse_core_info()
  nc, ns = info.num_cores, info.num_subcores
  nw = nc * ns
  lanes = 2 * hid
  nb_full = vocab // lanes
  tail = vocab - nb_full * lanes
  max_iters = (nb_full + nw - 1) // nw

  mesh = plsc.VectorSubcoreMesh(core_axis_name="c", subcore_axis_name="s")

  @functools.partial(
      pl.kernel,
      mesh=mesh,
      compiler_params=pltpu.CompilerParams(needs_layout_passes=False),
      out_type=jax.ShapeDtypeStruct((vocab * hid,), jnp.float32),
      scratch_types=[
          pltpu.VMEM((hid, lanes), jnp.float32),
          pltpu.VMEM((hid, lanes), jnp.float32),
          pltpu.VMEM((hid * lanes,), jnp.float32),
          pltpu.VMEM((hid * lanes,), jnp.float32),
          pltpu.VMEM((hid, hid), jnp.float32),
          pltpu.SemaphoreType.DMA,
          pltpu.SemaphoreType.DMA,
          pltpu.SemaphoreType.DMA,
          pltpu.SemaphoreType.DMA,
          pltpu.SemaphoreType.DMA,
      ],
  )
  def sc_t(tin, out, vbuf0, vbuf1, tbuf0, tbuf1, vtail,
           in0, in1, out0, out1, tsem):
    wid = lax.axis_index("s") * nc + lax.axis_index("c")
    n_my = (nb_full - wid + nw - 1) // nw
    vbufs = (vbuf0, vbuf1)
    tbufs = (tbuf0, tbuf1)
    in_sems = (in0, in1)
    out_sems = (out0, out1)
    f16 = lax.iota(jnp.int32, 16)

    def start_in(i):
      g = wid + i * nw
      b = lax.rem(i, 2)
      for bb in range(2):
        @pl.when(b == bb)
        def _():
          pltpu.async_copy(tin.at[:, pl.ds(g * lanes, lanes)],
                           vbufs[bb], in_sems[bb])

    def transpose_block(bb, n_cols, src, dst):
      @plsc.parallel_loop(0, n_cols, step=1, unroll=8)
      def col(v):
        vv = jnp.zeros((16,), jnp.int32) + v
        for kk in range(hid // 16):
          vec = plsc.load_gather(src, [f16 + kk * 16, vv])
          dst[pl.ds(v * hid + kk * 16, 16)] = vec

    @pl.when(n_my > 0)
    def _():
      start_in(0)

      def body(i, carry):
        g = wid + i * nw
        b = lax.rem(i, 2)
        for bb in range(2):
          @pl.when(b == bb)
          def _():
            pltpu.make_async_copy(tin.at[:, pl.ds(g * lanes, lanes)],
                                  vbufs[bb], in_sems[bb]).wait()

        @pl.when(i + 1 < n_my)
        def _():
          start_in(i + 1)

        for bb in range(2):
          @pl.when(b == bb)
          def _():
            @pl.when(i >= 2)
            def _():
              pltpu.make_async_copy(
                  tbufs[bb], out.at[pl.ds(g * hid * lanes, hid * lanes)],
                  out_sems[bb]).wait()
            transpose_block(bb, lanes, vbufs[bb], tbufs[bb])
            pltpu.async_copy(
                tbufs[bb], out.at[pl.ds(g * hid * lanes, hid * lanes)],
                out_sems[bb])
        return carry

      lax.fori_loop(0, n_my, body, 0)

      def drain(i, carry):
        b = lax.rem(i, 2)
        g = wid + i * nw
        for bb in range(2):
          @pl.when((b == bb) & (i + 2 >= n_my))
          def _():
            pltpu.make_async_copy(
                tbufs[bb], out.at[pl.ds(g * hid * lanes, hid * lanes)],
                out_sems[bb]).wait()
        return carry
      lax.fori_loop(jnp.maximum(n_my - 2, 0), n_my, drain, 0)

    if tail:
      @pl.when(wid == nw - 1)
      def _():
        pltpu.async_copy(tin.at[:, pl.ds(nb_full * lanes, tail)], vtail, tsem)
        pltpu.make_async_copy(tin.at[:, pl.ds(nb_full * lanes, tail)], vtail,
                              tsem).wait()
        @plsc.parallel_loop(0, tail, step=1, unroll=8)
        def col(v):
          vv = jnp.zeros((16,), jnp.int32) + v
          for kk in range(hid // 16):
            vec = plsc.load_gather(vtail, [f16 + kk * 16, vv])
            tbuf0[pl.ds(v * hid + kk * 16, 16)] = vec
        pltpu.async_copy(
            tbuf0.at[pl.ds(0, tail * hid)],
            out.at[pl.ds(nb_full * lanes * hid, tail * hid)], tsem)
        pltpu.make_async_copy(
            tbuf0.at[pl.ds(0, tail * hid)],
            out.at[pl.ds(nb_full * lanes * hid, tail * hid)], tsem).wait()

  return sc_t


def _tt_body(x_ref, out_ref):
  hid, vb = x_ref.shape
  y = jnp.transpose(x_ref[...], (1, 0))
  out_ref[...] = jnp.concatenate([y[:vb // 2, :], y[vb // 2:, :]], axis=1)


def _make_tc_transpose(vocab, hid, vb):
  n_blk = (vocab + vb - 1) // vb
  return pl.pallas_call(
      _tt_body,
      grid=(n_blk,),
      in_specs=[pl.BlockSpec((hid, vb), lambda i: (0, i))],
      out_specs=pl.BlockSpec((vb // 2, 2 * hid), lambda i: (i, 0)),
      out_shape=jax.ShapeDtypeStruct((n_blk * vb // 2, 2 * hid), jnp.float32),
  )


def _make_sc_gather(vocab, hid, n_rows, chunk):
  info = plsc.get_sparse_core_info()
  nc, ns = info.num_cores, info.num_subcores
  nw = nc * ns
  per_w = n_rows // nw
  assert n_rows % nw == 0 and per_w % chunk == 0
  n_chunks = per_w // chunk

  mesh = plsc.VectorSubcoreMesh(core_axis_name="c", subcore_axis_name="s")

  @functools.partial(
      pl.kernel,
      mesh=mesh,
      compiler_params=pltpu.CompilerParams(use_tc_tiling_on_sc=False),
      out_type=jax.ShapeDtypeStruct((n_rows, 2 * hid), jnp.float32),
      scratch_types=[
          pltpu.VMEM((chunk,), jnp.int32),
          pltpu.VMEM((chunk, hid), jnp.float32),
          pltpu.SemaphoreType.DMA,
      ],
  )
  def sc_gather(table_hbm, idx_hbm, out_hbm, idx_v, rows_v, sem):
    wid = lax.axis_index("s") * nc + lax.axis_index("c")
    w_base = wid * per_w

    def body(i, carry):
      base = w_base + i * chunk
      pltpu.sync_copy(idx_hbm.at[pl.ds(base, chunk)], idx_v)
      pltpu.async_copy(table_hbm.at[idx_v], rows_v, sem).wait()
      pltpu.sync_copy(rows_v, out_hbm.at[pl.ds(base, chunk), pl.ds(0, hid)])
      return carry

    lax.fori_loop(0, n_chunks, body, 0)

  return sc_gather


def _conv_body(x_ref, w_ref, b_ref, out_ref):
  bs, s, hid = x_ref.shape
  x = x_ref[...]
  zero = jnp.zeros((bs, 1, hid), jnp.float32)
  x_prev = jnp.concatenate([zero, x[:, :-1, :]], axis=1)
  x_next = jnp.concatenate([x[:, 1:, :], zero], axis=1)
  xcat = jnp.concatenate([x_prev, x, x_next], axis=2)
  y = jnp.dot(
      xcat.reshape(bs * s, 3 * hid), w_ref[...],
      preferred_element_type=jnp.float32)
  y = y.reshape(bs, s, hid)
  m = jnp.max(y, axis=1)
  out_ref[...] = jnp.maximum(m + b_ref[...], 0.0)


def _make_tc_conv(b, s, hid, bs_blk):
  assert b % bs_blk == 0
  grid = (b // bs_blk,)
  return pl.pallas_call(
      _conv_body,
      grid=grid,
      in_specs=[
          pl.BlockSpec((bs_blk, s, hid), lambda i: (i, 0, 0)),
          pl.BlockSpec((3 * hid, hid), lambda i: (0, 0)),
          pl.BlockSpec((1, hid), lambda i: (0, 0)),
      ],
      out_specs=pl.BlockSpec((bs_blk, hid), lambda i: (i, 0)),
      out_shape=jax.ShapeDtypeStruct((b, hid), jnp.float32),
  )


def kernel(input, table, conv_w, conv_b):
  b, s = input.shape
  vocab, hid = table.shape
  k = conv_w.shape[2]
  n_rows = b * s

  idx = input.reshape(n_rows)

  vb = 32768
  tc_t = _make_tc_transpose(vocab, hid, vb=vb)
  packed = tc_t(jnp.transpose(table, (1, 0)))
  table_rows = packed.reshape(packed.shape[0] * 2, hid)

  j = idx % vb
  idx2 = (idx // vb) * vb + (j % (vb // 2)) * 2 + j // (vb // 2)

  sc_gather = _make_sc_gather(vocab, hid, n_rows, chunk=1280)
  out_wide = sc_gather(table_rows, idx2)
  embed = out_wide.reshape(b, s, 2 * hid)[:, :, :hid]

  w_full = jnp.transpose(conv_w, (2, 1, 0)).reshape(k * hid, hid)
  tc_conv = _make_tc_conv(b, s, hid, bs_blk=128)
  hidden = tc_conv(embed, w_full, conv_b.reshape(1, hid))

  return (embed, hidden)


# speedup vs baseline: 2.9881x; 1.0099x over previous
import functools

import jax
import jax.numpy as jnp
from jax import lax
from jax.experimental import pallas as pl
from jax.experimental.pallas import tpu as pltpu
from jax.experimental.pallas import tpu_sc as plsc


def _make_sc_transpose(vocab, hid):
  info = plsc.get_sparse_core_info()
  nc, ns = info.num_cores, info.num_subcores
  nw = nc * ns
  lanes = 2 * hid
  nb_full = vocab // lanes
  tail = vocab - nb_full * lanes
  max_iters = (nb_full + nw - 1) // nw

  mesh = plsc.VectorSubcoreMesh(core_axis_name="c", subcore_axis_name="s")

  @functools.partial(
      pl.kernel,
      mesh=mesh,
      compiler_params=pltpu.CompilerParams(needs_layout_passes=False),
      out_type=jax.ShapeDtypeStruct((vocab * hid,), jnp.float32),
      scratch_types=[
          pltpu.VMEM((hid, lanes), jnp.float32),
          pltpu.VMEM((hid, lanes), jnp.float32),
          pltpu.VMEM((hid * lanes,), jnp.float32),
          pltpu.VMEM((hid * lanes,), jnp.float32),
          pltpu.VMEM((hid, hid), jnp.float32),
          pltpu.SemaphoreType.DMA,
          pltpu.SemaphoreType.DMA,
          pltpu.SemaphoreType.DMA,
          pltpu.SemaphoreType.DMA,
          pltpu.SemaphoreType.DMA,
      ],
  )
  def sc_t(tin, out, vbuf0, vbuf1, tbuf0, tbuf1, vtail,
           in0, in1, out0, out1, tsem):
    wid = lax.axis_index("s") * nc + lax.axis_index("c")
    n_my = (nb_full - wid + nw - 1) // nw
    vbufs = (vbuf0, vbuf1)
    tbufs = (tbuf0, tbuf1)
    in_sems = (in0, in1)
    out_sems = (out0, out1)
    f16 = lax.iota(jnp.int32, 16)

    def start_in(i):
      g = wid + i * nw
      b = lax.rem(i, 2)
      for bb in range(2):
        @pl.when(b == bb)
        def _():
          pltpu.async_copy(tin.at[:, pl.ds(g * lanes, lanes)],
                           vbufs[bb], in_sems[bb])

    def transpose_block(bb, n_cols, src, dst):
      @plsc.parallel_loop(0, n_cols, step=1, unroll=8)
      def col(v):
        vv = jnp.zeros((16,), jnp.int32) + v
        for kk in range(hid // 16):
          vec = plsc.load_gather(src, [f16 + kk * 16, vv])
          dst[pl.ds(v * hid + kk * 16, 16)] = vec

    @pl.when(n_my > 0)
    def _():
      start_in(0)

      def body(i, carry):
        g = wid + i * nw
        b = lax.rem(i, 2)
        for bb in range(2):
          @pl.when(b == bb)
          def _():
            pltpu.make_async_copy(tin.at[:, pl.ds(g * lanes, lanes)],
                                  vbufs[bb], in_sems[bb]).wait()

        @pl.when(i + 1 < n_my)
        def _():
          start_in(i + 1)

        for bb in range(2):
          @pl.when(b == bb)
          def _():
            @pl.when(i >= 2)
            def _():
              pltpu.make_async_copy(
                  tbufs[bb], out.at[pl.ds(g * hid * lanes, hid * lanes)],
                  out_sems[bb]).wait()
            transpose_block(bb, lanes, vbufs[bb], tbufs[bb])
            pltpu.async_copy(
                tbufs[bb], out.at[pl.ds(g * hid * lanes, hid * lanes)],
                out_sems[bb])
        return carry

      lax.fori_loop(0, n_my, body, 0)

      def drain(i, carry):
        b = lax.rem(i, 2)
        g = wid + i * nw
        for bb in range(2):
          @pl.when((b == bb) & (i + 2 >= n_my))
          def _():
            pltpu.make_async_copy(
                tbufs[bb], out.at[pl.ds(g * hid * lanes, hid * lanes)],
                out_sems[bb]).wait()
        return carry
      lax.fori_loop(jnp.maximum(n_my - 2, 0), n_my, drain, 0)

    if tail:
      @pl.when(wid == nw - 1)
      def _():
        pltpu.async_copy(tin.at[:, pl.ds(nb_full * lanes, tail)], vtail, tsem)
        pltpu.make_async_copy(tin.at[:, pl.ds(nb_full * lanes, tail)], vtail,
                              tsem).wait()
        @plsc.parallel_loop(0, tail, step=1, unroll=8)
        def col(v):
          vv = jnp.zeros((16,), jnp.int32) + v
          for kk in range(hid // 16):
            vec = plsc.load_gather(vtail, [f16 + kk * 16, vv])
            tbuf0[pl.ds(v * hid + kk * 16, 16)] = vec
        pltpu.async_copy(
            tbuf0.at[pl.ds(0, tail * hid)],
            out.at[pl.ds(nb_full * lanes * hid, tail * hid)], tsem)
        pltpu.make_async_copy(
            tbuf0.at[pl.ds(0, tail * hid)],
            out.at[pl.ds(nb_full * lanes * hid, tail * hid)], tsem).wait()

  return sc_t


def _tt_body(x_ref, out_ref):
  hid, vb = x_ref.shape
  y = jnp.transpose(x_ref[...], (1, 0))
  out_ref[...] = jnp.concatenate([y[:vb // 2, :], y[vb // 2:, :]], axis=1)


def _make_tc_transpose(vocab, hid, vb):
  n_blk = (vocab + vb - 1) // vb
  return pl.pallas_call(
      _tt_body,
      grid=(n_blk,),
      in_specs=[pl.BlockSpec((hid, vb), lambda i: (0, i))],
      out_specs=pl.BlockSpec((vb // 2, 2 * hid), lambda i: (i, 0)),
      out_shape=jax.ShapeDtypeStruct((n_blk * vb // 2, 2 * hid), jnp.float32),
  )


def _make_sc_gather(vocab, hid, n_rows, chunk):
  info = plsc.get_sparse_core_info()
  nc, ns = info.num_cores, info.num_subcores
  nw = nc * ns
  per_w = n_rows // nw
  assert n_rows % nw == 0 and per_w % chunk == 0
  n_chunks = per_w // chunk

  mesh = plsc.VectorSubcoreMesh(core_axis_name="c", subcore_axis_name="s")

  @functools.partial(
      pl.kernel,
      mesh=mesh,
      compiler_params=pltpu.CompilerParams(use_tc_tiling_on_sc=False),
      out_type=jax.ShapeDtypeStruct((n_rows, 2 * hid), jnp.float32),
      scratch_types=[
          pltpu.VMEM((chunk,), jnp.int32),
          pltpu.VMEM((chunk, hid), jnp.float32),
          pltpu.SemaphoreType.DMA,
      ],
  )
  def sc_gather(table_hbm, idx_hbm, out_hbm, idx_v, rows_v, sem):
    wid = lax.axis_index("s") * nc + lax.axis_index("c")
    w_base = wid * per_w

    def body(i, carry):
      base = w_base + i * chunk
      pltpu.sync_copy(idx_hbm.at[pl.ds(base, chunk)], idx_v)
      pltpu.async_copy(table_hbm.at[idx_v], rows_v, sem).wait()
      pltpu.sync_copy(rows_v, out_hbm.at[pl.ds(base, chunk), pl.ds(0, hid)])
      return carry

    lax.fori_loop(0, n_chunks, body, 0)

  return sc_gather


def _conv_body(x_ref, w_ref, b_ref, out_ref):
  bs, s, hid = x_ref.shape
  x = x_ref[...]
  zero = jnp.zeros((bs, 1, hid), jnp.float32)
  x_prev = jnp.concatenate([zero, x[:, :-1, :]], axis=1)
  x_next = jnp.concatenate([x[:, 1:, :], zero], axis=1)
  xcat = jnp.concatenate([x_prev, x, x_next], axis=2)
  y = jnp.dot(
      xcat.reshape(bs * s, 3 * hid), w_ref[...],
      preferred_element_type=jnp.float32)
  y = y.reshape(bs, s, hid)
  m = jnp.max(y, axis=1)
  out_ref[...] = jnp.maximum(m + b_ref[...], 0.0)


def _make_tc_conv(b, s, hid, bs_blk):
  assert b % bs_blk == 0
  grid = (b // bs_blk,)
  return pl.pallas_call(
      _conv_body,
      grid=grid,
      in_specs=[
          pl.BlockSpec((bs_blk, s, hid), lambda i: (i, 0, 0)),
          pl.BlockSpec((3 * hid, hid), lambda i: (0, 0)),
          pl.BlockSpec((1, hid), lambda i: (0, 0)),
      ],
      out_specs=pl.BlockSpec((bs_blk, hid), lambda i: (i, 0)),
      out_shape=jax.ShapeDtypeStruct((b, hid), jnp.float32),
  )


def kernel(input, table, conv_w, conv_b):
  b, s = input.shape
  vocab, hid = table.shape
  k = conv_w.shape[2]
  n_rows = b * s

  idx = input.reshape(n_rows)

  vb = 32768
  tc_t = _make_tc_transpose(vocab, hid, vb=vb)
  packed = tc_t(jnp.transpose(table, (1, 0)))
  table_rows = packed.reshape(packed.shape[0] * 2, hid)

  j = idx % vb
  idx2 = (idx // vb) * vb + (j % (vb // 2)) * 2 + j // (vb // 2)

  sc_gather = _make_sc_gather(vocab, hid, n_rows, chunk=1600)
  out_wide = sc_gather(table_rows, idx2)
  embed = out_wide.reshape(b, s, 2 * hid)[:, :, :hid]

  w_full = jnp.transpose(conv_w, (2, 1, 0)).reshape(k * hid, hid)
  tc_conv = _make_tc_conv(b, s, hid, bs_blk=128)
  hidden = tc_conv(embed, w_full, conv_b.reshape(1, hid))

  return (embed, hidden)
